# Initial kernel scaffold; baseline (speedup 1.0000x reference)
import jax
import jax.numpy as jnp
from jax.experimental import pallas as pl


def kernel(x, W_qkv, W_dw, temperature, W_proj):
    raise NotImplementedError("write your pallas kernel here")


# speedup vs baseline: 163.5045x; 163.5045x over previous
import jax
import jax.numpy as jnp
import numpy as np
from jax import lax
from jax.experimental import pallas as pl

B = 2
C = 384
HH = 16
WW = 16
GROUP = 4
HEADS = 8
CG = C // GROUP
CP = CG // HEADS
HW = HH * WW
N = HW * GROUP
KK = int(N * 0.8)
INT_MIN = np.int32(-(2 ** 31))


def _qkv_kernel(x_ref, wq_ref, wdw_ref, out_ref):
    xb = x_ref[0]
    wq = wq_ref[...]
    y = lax.dot_general(wq, xb, (((1,), (0,)), ((), ())),
                        preferred_element_type=jnp.float32)
    col = lax.broadcasted_iota(jnp.int32, (1, CG * HW), 1)
    hw = col % HW
    hh = hw // WW
    ww = hw % WW
    acc = jnp.zeros_like(y)
    for u in range(3):
        for v in range(3):
            dy = u - 1
            dx = v - 1
            d = dy * WW + dx
            src = y if d == 0 else jnp.roll(y, -d, axis=1)
            m = ((hh + dy >= 0) & (hh + dy < HH)
                 & (ww + dx >= 0) & (ww + dx < WW))
            tap = wdw_ref[:, 3 * u + v:3 * u + v + 1]
            acc = acc + jnp.where(m, src, 0.0) * tap
    out_ref[0] = acc


def _key_to_float(key):
    raw = jnp.where(key >= 0, key, ~(key ^ INT_MIN))
    return lax.bitcast_convert_type(raw, jnp.float32)


def _attn_kernel(q_ref, k_ref, v_ref, t_ref, out_ref):
    q = q_ref[0, 0]
    k = k_ref[0, 0]
    v = v_ref[0, 0]
    t = t_ref[0, 0, 0]
    qn = q / jnp.maximum(jnp.sqrt(jnp.sum(q * q, axis=1, keepdims=True)), 1e-12)
    kn = k / jnp.maximum(jnp.sqrt(jnp.sum(k * k, axis=1, keepdims=True)), 1e-12)
    a = lax.dot_general(qn, kn, (((0,), (0,)), ((), ())),
                        preferred_element_type=jnp.float32) * t

    def body(i, r):
        bit = 31 - i
        cand = r | (jnp.int32(1) << bit)
        thr = _key_to_float(cand ^ INT_MIN)
        cnt = jnp.sum(jnp.where(a >= thr, 1.0, 0.0), axis=1, keepdims=True)
        return jnp.where(cnt >= float(KK), cand, r)

    r = lax.fori_loop(0, 32, body, jnp.zeros((N, 1), jnp.int32))
    thr = _key_to_float(r ^ INT_MIN)

    rowmax = jnp.max(a, axis=1, keepdims=True)
    p = jnp.where(a >= thr, jnp.exp(a - rowmax), 0.0)
    p = p / jnp.sum(p, axis=1, keepdims=True)
    o = lax.dot_general(v, p, (((1,), (1,)), ((), ())),
                        preferred_element_type=jnp.float32)
    out_ref[0, 0] = o


def _proj_kernel(o_ref, x_ref, w_ref, out_ref):
    ob = o_ref[0]
    xb = x_ref[0]
    w = w_ref[...]
    out_ref[0] = xb + lax.dot_general(w, ob, (((1,), (0,)), ((), ())),
                                      preferred_element_type=jnp.float32)


def kernel(x, W_qkv, W_dw, temperature, W_proj):
    xg = x.reshape(B, GROUP, CG * HW)
    wdw9 = W_dw.reshape(3 * GROUP, 9)

    qkv2d = pl.pallas_call(
        _qkv_kernel,
        grid=(B,),
        in_specs=[
            pl.BlockSpec((1, GROUP, CG * HW), lambda b: (b, 0, 0)),
            pl.BlockSpec((3 * GROUP, GROUP), lambda b: (0, 0)),
            pl.BlockSpec((3 * GROUP, 9), lambda b: (0, 0)),
        ],
        out_specs=pl.BlockSpec((1, 3 * GROUP, CG * HW), lambda b: (b, 0, 0)),
        out_shape=jax.ShapeDtypeStruct((B, 3 * GROUP, CG * HW), jnp.float32),
    )(xg, W_qkv, wdw9)

    arr = (qkv2d.reshape(B, 3, GROUP, HEADS, CP, HW)
           .transpose(0, 1, 3, 4, 2, 5)
           .reshape(B, 3, HEADS, CP, N))
    qh, kh, vh = arr[:, 0], arr[:, 1], arr[:, 2]
    tb = jnp.broadcast_to(temperature.reshape(HEADS, 1, 1).astype(jnp.float32),
                          (HEADS, 1, 128))

    head_spec = pl.BlockSpec((1, 1, CP, N), lambda b, h: (b, h, 0, 0))
    oh = pl.pallas_call(
        _attn_kernel,
        grid=(B, HEADS),
        in_specs=[head_spec, head_spec, head_spec,
                  pl.BlockSpec((1, 1, 128), lambda b, h: (h, 0, 0))],
        out_specs=pl.BlockSpec((1, 1, CP, N), lambda b, h: (b, h, 0, 0)),
        out_shape=jax.ShapeDtypeStruct((B, HEADS, CP, N), jnp.float32),
    )(qh, kh, vh, tb)

    o2 = (oh.reshape(B, HEADS, CP, GROUP, HW)
          .transpose(0, 3, 1, 2, 4)
          .reshape(B, C, HW))
    x2 = x.reshape(B, C, HW)

    y = pl.pallas_call(
        _proj_kernel,
        grid=(B,),
        in_specs=[
            pl.BlockSpec((1, C, HW), lambda b: (b, 0, 0)),
            pl.BlockSpec((1, C, HW), lambda b: (b, 0, 0)),
            pl.BlockSpec((C, C), lambda b: (0, 0)),
        ],
        out_specs=pl.BlockSpec((1, C, HW), lambda b: (b, 0, 0)),
        out_shape=jax.ShapeDtypeStruct((B, C, HW), jnp.float32),
    )(o2, x2, W_proj)
    return y.reshape(B, C, HH, WW)


# speedup vs baseline: 399.7624x; 2.4450x over previous
import jax
import jax.numpy as jnp
import numpy as np
from jax import lax
from jax.experimental import pallas as pl

B = 2
C = 384
HH = 16
WW = 16
GROUP = 4
HEADS = 8
CG = C // GROUP
CP = CG // HEADS
HW = HH * WW
N = HW * GROUP
KK = int(N * 0.8)
RADIX_PASSES = 14
INT_MIN = np.int32(-(2 ** 31))


def _qkv_kernel(x_ref, wq_ref, wdw_ref, out_ref):
    col = lax.broadcasted_iota(jnp.int32, (1, HW), 1)
    hh = col // WW
    ww = col % WW
    xs = [x_ref[0, tin * CG:(tin + 1) * CG, :] for tin in range(GROUP)]
    for part in range(3):
        for ti in range(GROUP):
            o = part * GROUP + ti
            y = xs[0] * wq_ref[o:o + 1, 0:1]
            for tin in range(1, GROUP):
                y = y + xs[tin] * wq_ref[o:o + 1, tin:tin + 1]
            acc = None
            for u in range(3):
                for v in range(3):
                    dy = u - 1
                    dx = v - 1
                    d = dy * WW + dx
                    src = y if d == 0 else jnp.roll(y, -d, axis=1)
                    m = ((hh + dy >= 0) & (hh + dy < HH)
                         & (ww + dx >= 0) & (ww + dx < WW))
                    term = jnp.where(m, src, 0.0) * wdw_ref[o:o + 1, 3 * u + v:3 * u + v + 1]
                    acc = term if acc is None else acc + term
            out_ref[0, pl.ds(part * CG, CG), pl.ds(ti * HW, HW)] = acc


def _key_to_float(key):
    raw = jnp.where(key >= 0, key, ~(key ^ INT_MIN))
    return lax.bitcast_convert_type(raw, jnp.float32)


def _attn_kernel(q_ref, k_ref, v_ref, t_ref, out_ref):
    q = q_ref[0, 0]
    k = k_ref[0, 0]
    v = v_ref[0, 0]
    t = t_ref[0, 0, 0]
    qn = q * (t / jnp.maximum(jnp.sqrt(jnp.sum(q * q, axis=1, keepdims=True)), 1e-12))
    kn = k / jnp.maximum(jnp.sqrt(jnp.sum(k * k, axis=1, keepdims=True)), 1e-12)
    at = lax.dot_general(kn, qn, (((0,), (0,)), ((), ())),
                         preferred_element_type=jnp.float32)

    def body(i, r):
        bit = 31 - i
        cand = r | (jnp.int32(1) << bit)
        thr = _key_to_float(cand ^ INT_MIN)
        cnt = jnp.sum(jnp.where(at >= thr, 1.0, 0.0), axis=0, keepdims=True)
        return jnp.where(cnt >= float(KK), cand, r)

    r = lax.fori_loop(0, RADIX_PASSES, body, jnp.zeros((1, N), jnp.int32))
    thr = _key_to_float(r ^ INT_MIN)

    p = jnp.where(at >= thr, jnp.exp(at), 0.0)
    zinv = 1.0 / jnp.sum(p, axis=0, keepdims=True)
    for ti in range(GROUP):
        psl = p[:, ti * HW:(ti + 1) * HW]
        osl = lax.dot_general(v, psl, (((1,), (0,)), ((), ())),
                              preferred_element_type=jnp.float32)
        out_ref[0, ti, 0] = osl * zinv[0:1, ti * HW:(ti + 1) * HW]


def _proj_kernel(o_ref, x_ref, w_ref, out_ref):
    ob = o_ref[0]
    xb = x_ref[0]
    w = w_ref[...]
    out_ref[0] = xb + lax.dot_general(w, ob, (((1,), (0,)), ((), ())),
                                      preferred_element_type=jnp.float32)


def kernel(x, W_qkv, W_dw, temperature, W_proj):
    x2 = x.reshape(B, C, HW)
    wdw9 = W_dw.reshape(3 * GROUP, 9)

    qkv = pl.pallas_call(
        _qkv_kernel,
        grid=(B,),
        in_specs=[
            pl.BlockSpec((1, C, HW), lambda b: (b, 0, 0)),
            pl.BlockSpec((3 * GROUP, GROUP), lambda b: (0, 0)),
            pl.BlockSpec((3 * GROUP, 9), lambda b: (0, 0)),
        ],
        out_specs=pl.BlockSpec((1, 3 * CG, N), lambda b: (b, 0, 0)),
        out_shape=jax.ShapeDtypeStruct((B, 3 * CG, N), jnp.float32),
    )(x2, W_qkv, wdw9)

    arr = qkv.reshape(B, 3, HEADS, CP, N)
    qh, kh, vh = arr[:, 0], arr[:, 1], arr[:, 2]
    tb = jnp.broadcast_to(temperature.reshape(HEADS, 1, 1).astype(jnp.float32),
                          (HEADS, 1, 128))

    head_spec = pl.BlockSpec((1, 1, CP, N), lambda b, h: (b, h, 0, 0))
    oh = pl.pallas_call(
        _attn_kernel,
        grid=(B, HEADS),
        in_specs=[head_spec, head_spec, head_spec,
                  pl.BlockSpec((1, 1, 128), lambda b, h: (h, 0, 0))],
        out_specs=pl.BlockSpec((1, GROUP, 1, CP, HW),
                               lambda b, h: (b, 0, h, 0, 0)),
        out_shape=jax.ShapeDtypeStruct((B, GROUP, HEADS, CP, HW), jnp.float32),
    )(qh, kh, vh, tb)

    o2 = oh.reshape(B, C, HW)
    y = pl.pallas_call(
        _proj_kernel,
        grid=(B,),
        in_specs=[
            pl.BlockSpec((1, C, HW), lambda b: (b, 0, 0)),
            pl.BlockSpec((1, C, HW), lambda b: (b, 0, 0)),
            pl.BlockSpec((C, C), lambda b: (0, 0)),
        ],
        out_specs=pl.BlockSpec((1, C, HW), lambda b: (b, 0, 0)),
        out_shape=jax.ShapeDtypeStruct((B, C, HW), jnp.float32),
    )(o2, x2, W_proj)
    return y.reshape(B, C, HH, WW)


# speedup vs baseline: 540.3839x; 1.3518x over previous
import jax
import jax.numpy as jnp
import numpy as np
from jax import lax
from jax.experimental import pallas as pl

B = 2
C = 384
HH = 16
WW = 16
GROUP = 4
HEADS = 8
CG = C // GROUP
CP = CG // HEADS
HW = HH * WW
N = HW * GROUP
KK = int(N * 0.8)
RADIX_PASSES = 14
INT_MIN = np.int32(-(2 ** 31))


def _qkv_kernel(x_ref, wq_ref, wdw_ref, out_ref):
    col = lax.broadcasted_iota(jnp.int32, (1, HW), 1)
    hh = col // WW
    ww = col % WW
    xs = [x_ref[0, tin * CG:(tin + 1) * CG, :] for tin in range(GROUP)]
    for part in range(3):
        for ti in range(GROUP):
            o = part * GROUP + ti
            y = xs[0] * wq_ref[o:o + 1, 0:1]
            for tin in range(1, GROUP):
                y = y + xs[tin] * wq_ref[o:o + 1, tin:tin + 1]
            acc = None
            for u in range(3):
                for v in range(3):
                    dy = u - 1
                    dx = v - 1
                    d = dy * WW + dx
                    src = y if d == 0 else jnp.roll(y, -d, axis=1)
                    m = ((hh + dy >= 0) & (hh + dy < HH)
                         & (ww + dx >= 0) & (ww + dx < WW))
                    term = jnp.where(m, src, 0.0) * wdw_ref[o:o + 1, 3 * u + v:3 * u + v + 1]
                    acc = term if acc is None else acc + term
            out_ref[0, pl.ds(part * CG, CG), pl.ds(ti * HW, HW)] = acc


def _key_to_float(key):
    raw = jnp.where(key >= 0, key, ~(key ^ INT_MIN))
    return lax.bitcast_convert_type(raw, jnp.float32)


def _attn_kernel(q_ref, k_ref, v_ref, t_ref, out_ref):
    q = q_ref[0, 0]
    k = k_ref[0, 0]
    v = v_ref[0, 0]
    t = t_ref[0, 0, 0]
    qn = q * (t / jnp.maximum(jnp.sqrt(jnp.sum(q * q, axis=1, keepdims=True)), 1e-12))
    kn = k / jnp.maximum(jnp.sqrt(jnp.sum(k * k, axis=1, keepdims=True)), 1e-12)
    at = lax.dot_general(kn, qn, (((0,), (0,)), ((), ())),
                         preferred_element_type=jnp.float32)

    ai = lax.bitcast_convert_type(at, jnp.int32)
    ks = jnp.where(ai >= 0, ai, ~ai ^ INT_MIN)
    k16 = lax.shift_right_arithmetic(ks, 16).astype(jnp.int16)

    def body(i, carry):
        r, bm = carry
        cand = r | bm
        s16 = (cand - 32768).astype(jnp.int16)
        m = (k16 >= s16).astype(jnp.int16)
        for sz in (512, 256, 128, 64, 32, 16):
            m = m[:sz] + m[sz:]
        cnt = jnp.sum(m.astype(jnp.int32), axis=0, keepdims=True)
        r = jnp.where(cnt >= KK, cand, r)
        return (r, lax.shift_right_logical(bm, 1))

    r, _ = lax.fori_loop(
        0, RADIX_PASSES, body,
        (jnp.zeros((1, N), jnp.int32), jnp.int32(2 ** 15)))
    key_prefix = lax.shift_left(r - 32768, 16)
    thr = _key_to_float(key_prefix)

    p = jnp.where(at >= thr, jnp.exp(at), 0.0)
    vo = jnp.concatenate([v, jnp.ones((1, N), jnp.float32)], axis=0)
    for ti in range(GROUP):
        psl = p[:, ti * HW:(ti + 1) * HW]
        osl = lax.dot_general(vo, psl, (((1,), (0,)), ((), ())),
                              preferred_element_type=jnp.float32)
        out_ref[0, ti, 0] = osl[:CP, :] / osl[CP:CP + 1, :]


def _proj_kernel(o_ref, x_ref, w_ref, out_ref):
    ob = o_ref[0]
    xb = x_ref[0]
    w = w_ref[...]
    out_ref[0] = xb + lax.dot_general(w, ob, (((1,), (0,)), ((), ())),
                                      preferred_element_type=jnp.float32)


def kernel(x, W_qkv, W_dw, temperature, W_proj):
    x2 = x.reshape(B, C, HW)
    wdw9 = W_dw.reshape(3 * GROUP, 9)

    qkv = pl.pallas_call(
        _qkv_kernel,
        grid=(B,),
        in_specs=[
            pl.BlockSpec((1, C, HW), lambda b: (b, 0, 0)),
            pl.BlockSpec((3 * GROUP, GROUP), lambda b: (0, 0)),
            pl.BlockSpec((3 * GROUP, 9), lambda b: (0, 0)),
        ],
        out_specs=pl.BlockSpec((1, 3 * CG, N), lambda b: (b, 0, 0)),
        out_shape=jax.ShapeDtypeStruct((B, 3 * CG, N), jnp.float32),
    )(x2, W_qkv, wdw9)

    arr = qkv.reshape(B, 3, HEADS, CP, N)
    qh, kh, vh = arr[:, 0], arr[:, 1], arr[:, 2]
    tb = jnp.broadcast_to(temperature.reshape(HEADS, 1, 1).astype(jnp.float32),
                          (HEADS, 1, 128))

    head_spec = pl.BlockSpec((1, 1, CP, N), lambda b, h: (b, h, 0, 0))
    oh = pl.pallas_call(
        _attn_kernel,
        grid=(B, HEADS),
        in_specs=[head_spec, head_spec, head_spec,
                  pl.BlockSpec((1, 1, 128), lambda b, h: (h, 0, 0))],
        out_specs=pl.BlockSpec((1, GROUP, 1, CP, HW),
                               lambda b, h: (b, 0, h, 0, 0)),
        out_shape=jax.ShapeDtypeStruct((B, GROUP, HEADS, CP, HW), jnp.float32),
    )(qh, kh, vh, tb)

    o2 = oh.reshape(B, C, HW)
    y = pl.pallas_call(
        _proj_kernel,
        grid=(B,),
        in_specs=[
            pl.BlockSpec((1, C, HW), lambda b: (b, 0, 0)),
            pl.BlockSpec((1, C, HW), lambda b: (b, 0, 0)),
            pl.BlockSpec((C, C), lambda b: (0, 0)),
        ],
        out_specs=pl.BlockSpec((1, C, HW), lambda b: (b, 0, 0)),
        out_shape=jax.ShapeDtypeStruct((B, C, HW), jnp.float32),
    )(o2, x2, W_proj)
    return y.reshape(B, C, HH, WW)


# speedup vs baseline: 633.6027x; 1.1725x over previous
import jax
import jax.numpy as jnp
import numpy as np
from jax import lax
from jax.experimental import pallas as pl
from jax.experimental.pallas import tpu as pltpu

B = 2
C = 384
HH = 16
WW = 16
GROUP = 4
HEADS = 8
CG = C // GROUP
CP = CG // HEADS
HW = HH * WW
N = HW * GROUP
KK = int(N * 0.8)
RADIX_PASSES = 12
INT_MIN = np.int32(-(2 ** 31))


def _key_to_float(key):
    raw = jnp.where(key >= 0, key, ~(key ^ INT_MIN))
    return lax.bitcast_convert_type(raw, jnp.float32)


def _fused_kernel(xh_ref, x2_ref, wq_ref, wdw_ref, t_ref, wp_ref,
                  out_ref, o_acc):
    h = pl.program_id(1)
    t = t_ref[0, 0, 0]

    xh = [xh_ref[0, tin, 0] for tin in range(GROUP)]
    col = lax.broadcasted_iota(jnp.int32, (1, HW), 1)
    hh = col // WW
    ww = col % WW
    parts = []
    for part in range(3):
        slabs = []
        for ti in range(GROUP):
            o = part * GROUP + ti
            y = xh[0] * wq_ref[o:o + 1, 0:1]
            for tin in range(1, GROUP):
                y = y + xh[tin] * wq_ref[o:o + 1, tin:tin + 1]
            acc = None
            for u in range(3):
                for v in range(3):
                    dy = u - 1
                    dx = v - 1
                    d = dy * WW + dx
                    src = y if d == 0 else jnp.roll(y, -d, axis=1)
                    m = ((hh + dy >= 0) & (hh + dy < HH)
                         & (ww + dx >= 0) & (ww + dx < WW))
                    term = (jnp.where(m, src, 0.0)
                            * wdw_ref[o:o + 1, 3 * u + v:3 * u + v + 1])
                    acc = term if acc is None else acc + term
            slabs.append(acc)
        parts.append(jnp.concatenate(slabs, axis=1))
    q, k, v = parts

    qn = q * (t / jnp.maximum(jnp.sqrt(jnp.sum(q * q, axis=1, keepdims=True)), 1e-12))
    kn = k / jnp.maximum(jnp.sqrt(jnp.sum(k * k, axis=1, keepdims=True)), 1e-12)
    at = lax.dot_general(kn.astype(jnp.bfloat16), qn.astype(jnp.bfloat16),
                         (((0,), (0,)), ((), ())),
                         preferred_element_type=jnp.float32)

    ah = lax.shift_right_arithmetic(lax.bitcast_convert_type(at, jnp.int32),
                                    16).astype(jnp.int16)
    k16 = jnp.where(ah >= 0, ah, ah ^ jnp.full_like(ah, 0x7FFF))

    def body(i, carry):
        r, bm = carry
        cand = r | bm
        s16 = (cand - 32768).astype(jnp.int16)
        m = (k16 >= s16).astype(jnp.int16)
        for sz in (512, 256, 128, 64, 32, 16):
            m = m[:sz] + m[sz:]
        cnt = jnp.sum(m.astype(jnp.int32), axis=0, keepdims=True)
        r = jnp.where(cnt >= KK, cand, r)
        return (r, lax.shift_right_logical(bm, 1))

    r, _ = lax.fori_loop(
        0, RADIX_PASSES, body,
        (jnp.zeros((1, N), jnp.int32), jnp.int32(2 ** 15)))
    key_prefix = lax.shift_left(r - 32768, 16)
    thr = _key_to_float(key_prefix)

    p = jnp.where(at >= thr, jnp.exp(at), 0.0)
    vo = jnp.concatenate([v, jnp.ones((1, N), jnp.float32)], axis=0)
    for ti in range(GROUP):
        psl = p[:, ti * HW:(ti + 1) * HW]
        osl = lax.dot_general(vo, psl, (((1,), (0,)), ((), ())),
                              preferred_element_type=jnp.float32)
        o_acc[ti, h] = osl[:CP, :] / osl[CP:CP + 1, :]

    @pl.when(h == HEADS - 1)
    def _project():
        o2 = o_acc[...].reshape(C, HW)
        out_ref[0] = x2_ref[0] + lax.dot_general(
            wp_ref[...], o2, (((1,), (0,)), ((), ())),
            preferred_element_type=jnp.float32)


def kernel(x, W_qkv, W_dw, temperature, W_proj):
    x2 = x.reshape(B, C, HW)
    xh4 = x.reshape(B, GROUP, HEADS, CP, HW)
    wdw9 = W_dw.reshape(3 * GROUP, 9)
    tb = jnp.broadcast_to(temperature.reshape(HEADS, 1, 1).astype(jnp.float32),
                          (HEADS, 1, 128))

    y = pl.pallas_call(
        _fused_kernel,
        grid=(B, HEADS),
        in_specs=[
            pl.BlockSpec((1, GROUP, 1, CP, HW), lambda b, h: (b, 0, h, 0, 0)),
            pl.BlockSpec((1, C, HW), lambda b, h: (b, 0, 0)),
            pl.BlockSpec((3 * GROUP, GROUP), lambda b, h: (0, 0)),
            pl.BlockSpec((3 * GROUP, 9), lambda b, h: (0, 0)),
            pl.BlockSpec((1, 1, 128), lambda b, h: (h, 0, 0)),
            pl.BlockSpec((C, C), lambda b, h: (0, 0)),
        ],
        out_specs=pl.BlockSpec((1, C, HW), lambda b, h: (b, 0, 0)),
        out_shape=jax.ShapeDtypeStruct((B, C, HW), jnp.float32),
        scratch_shapes=[pltpu.VMEM((GROUP, HEADS, CP, HW), jnp.float32)],
    )(xh4, x2, W_qkv, wdw9, tb, W_proj)
    return y.reshape(B, C, HH, WW)


# speedup vs baseline: 683.9241x; 1.0794x over previous
import jax
import jax.numpy as jnp
import numpy as np
from jax import lax
from jax.experimental import pallas as pl
from jax.experimental.pallas import tpu as pltpu

B = 2
C = 384
HH = 16
WW = 16
GROUP = 4
HEADS = 8
CG = C // GROUP
CP = CG // HEADS
HW = HH * WW
N = HW * GROUP
KK = int(N * 0.8)
RADIX_PASSES = 10
INT_MIN = np.int32(-(2 ** 31))


def _key_to_float(key):
    raw = jnp.where(key >= 0, key, ~(key ^ INT_MIN))
    return lax.bitcast_convert_type(raw, jnp.float32)


def _fused_kernel(xh_ref, x2_ref, wq_ref, wdw_ref, t_ref, wp_ref,
                  out_ref, o_acc):
    h = pl.program_id(1)
    t = t_ref[0, 0, 0]

    xh = [xh_ref[0, tin, 0] for tin in range(GROUP)]
    col = lax.broadcasted_iota(jnp.int32, (1, HW), 1)
    hh = col // WW
    ww = col % WW
    parts = []
    for part in range(3):
        slabs = []
        for ti in range(GROUP):
            o = part * GROUP + ti
            y = xh[0] * wq_ref[o:o + 1, 0:1]
            for tin in range(1, GROUP):
                y = y + xh[tin] * wq_ref[o:o + 1, tin:tin + 1]
            acc = None
            for u in range(3):
                for v in range(3):
                    dy = u - 1
                    dx = v - 1
                    d = dy * WW + dx
                    src = y if d == 0 else jnp.roll(y, -d, axis=1)
                    m = ((hh + dy >= 0) & (hh + dy < HH)
                         & (ww + dx >= 0) & (ww + dx < WW))
                    term = (jnp.where(m, src, 0.0)
                            * wdw_ref[o:o + 1, 3 * u + v:3 * u + v + 1])
                    acc = term if acc is None else acc + term
            slabs.append(acc)
        parts.append(jnp.concatenate(slabs, axis=1))
    q, k, v = parts

    qn = q * (t / jnp.maximum(jnp.sqrt(jnp.sum(q * q, axis=1, keepdims=True)), 1e-12))
    kn = k / jnp.maximum(jnp.sqrt(jnp.sum(k * k, axis=1, keepdims=True)), 1e-12)
    at = lax.dot_general(kn.astype(jnp.bfloat16), qn.astype(jnp.bfloat16),
                         (((0,), (0,)), ((), ())),
                         preferred_element_type=jnp.float32)

    ah = lax.shift_right_arithmetic(lax.bitcast_convert_type(at, jnp.int32),
                                    16).astype(jnp.int16)
    k16 = jnp.where(ah >= 0, ah, ah ^ jnp.full_like(ah, 0x7FFF))

    r = jnp.zeros((1, N), jnp.int32)
    for i in range(RADIX_PASSES):
        cand = r | (1 << (15 - i))
        s16 = (cand - 32768).astype(jnp.int16)
        m = (k16 >= s16).astype(jnp.int16)
        for sz in (512, 256, 128, 64, 32, 16):
            m = m[:sz] + m[sz:]
        cnt = jnp.sum(m.astype(jnp.int32), axis=0, keepdims=True)
        r = jnp.where(cnt >= KK, cand, r)
    key_prefix = lax.shift_left(r - 32768, 16)
    thr = _key_to_float(key_prefix)

    p = jnp.where(at >= thr, jnp.exp(at), 0.0)
    vo = jnp.concatenate([v, jnp.ones((1, N), jnp.float32)], axis=0)
    for ti in range(GROUP):
        psl = p[:, ti * HW:(ti + 1) * HW]
        osl = lax.dot_general(vo, psl, (((1,), (0,)), ((), ())),
                              preferred_element_type=jnp.float32)
        o_acc[ti, h] = osl[:CP, :] / osl[CP:CP + 1, :]

    @pl.when(h == HEADS - 1)
    def _project():
        o2 = o_acc[...].reshape(C, HW)
        out_ref[0] = x2_ref[0] + lax.dot_general(
            wp_ref[...], o2, (((1,), (0,)), ((), ())),
            preferred_element_type=jnp.float32)


def kernel(x, W_qkv, W_dw, temperature, W_proj):
    x2 = x.reshape(B, C, HW)
    xh4 = x.reshape(B, GROUP, HEADS, CP, HW)
    wdw9 = W_dw.reshape(3 * GROUP, 9)
    tb = jnp.broadcast_to(temperature.reshape(HEADS, 1, 1).astype(jnp.float32),
                          (HEADS, 1, 128))

    y = pl.pallas_call(
        _fused_kernel,
        grid=(B, HEADS),
        in_specs=[
            pl.BlockSpec((1, GROUP, 1, CP, HW), lambda b, h: (b, 0, h, 0, 0)),
            pl.BlockSpec((1, C, HW), lambda b, h: (b, 0, 0)),
            pl.BlockSpec((3 * GROUP, GROUP), lambda b, h: (0, 0)),
            pl.BlockSpec((3 * GROUP, 9), lambda b, h: (0, 0)),
            pl.BlockSpec((1, 1, 128), lambda b, h: (h, 0, 0)),
            pl.BlockSpec((C, C), lambda b, h: (0, 0)),
        ],
        out_specs=pl.BlockSpec((1, C, HW), lambda b, h: (b, 0, 0)),
        out_shape=jax.ShapeDtypeStruct((B, C, HW), jnp.float32),
        scratch_shapes=[pltpu.VMEM((GROUP, HEADS, CP, HW), jnp.float32)],
    )(xh4, x2, W_qkv, wdw9, tb, W_proj)
    return y.reshape(B, C, HH, WW)


# speedup vs baseline: 704.3318x; 1.0298x over previous
import jax
import jax.numpy as jnp
import numpy as np
from jax import lax
from jax.experimental import pallas as pl
from jax.experimental.pallas import tpu as pltpu

B = 2
C = 384
HH = 16
WW = 16
GROUP = 4
HEADS = 8
CG = C // GROUP
CP = CG // HEADS
HW = HH * WW
N = HW * GROUP
KK = int(N * 0.8)
RADIX_PASSES = 10
INT_MIN = np.int32(-(2 ** 31))


def _key_to_float(key):
    raw = jnp.where(key >= 0, key, ~(key ^ INT_MIN))
    return lax.bitcast_convert_type(raw, jnp.float32)


def _fused_kernel(xh_ref, x2_ref, wq_ref, wdw_ref, t_ref, wp_ref,
                  out_ref, o_acc):
    h = pl.program_id(1)
    t = t_ref[0, h, 0, 0]

    xh = [xh_ref[0, tin, 0] for tin in range(GROUP)]
    col = lax.broadcasted_iota(jnp.int32, (1, HW), 1)
    hh = col // WW
    ww = col % WW
    parts = []
    for part in range(3):
        slabs = []
        for ti in range(GROUP):
            o = part * GROUP + ti
            y = xh[0] * wq_ref[o, 0]
            for tin in range(1, GROUP):
                y = y + xh[tin] * wq_ref[o, tin]
            acc = None
            for u in range(3):
                for v in range(3):
                    dy = u - 1
                    dx = v - 1
                    d = dy * WW + dx
                    src = y if d == 0 else jnp.roll(y, -d, axis=1)
                    m = ((hh + dy >= 0) & (hh + dy < HH)
                         & (ww + dx >= 0) & (ww + dx < WW))
                    term = jnp.where(m, src, 0.0) * wdw_ref[o, 0, u, v]
                    acc = term if acc is None else acc + term
            slabs.append(acc)
        parts.append(jnp.concatenate(slabs, axis=1))
    q, k, v = parts

    qn = q * (t / jnp.maximum(jnp.sqrt(jnp.sum(q * q, axis=1, keepdims=True)), 1e-12))
    kn = k / jnp.maximum(jnp.sqrt(jnp.sum(k * k, axis=1, keepdims=True)), 1e-12)
    at = lax.dot_general(kn.astype(jnp.bfloat16), qn.astype(jnp.bfloat16),
                         (((0,), (0,)), ((), ())),
                         preferred_element_type=jnp.float32)

    ah = lax.shift_right_arithmetic(lax.bitcast_convert_type(at, jnp.int32),
                                    16).astype(jnp.int16)
    k16 = jnp.where(ah >= 0, ah, ah ^ jnp.full_like(ah, 0x7FFF))
    r = jnp.zeros((1, N), jnp.int32)
    for i in range(RADIX_PASSES):
        cand = r | (1 << (15 - i))
        s16 = (cand - 32768).astype(jnp.int16)
        m = (k16 >= s16).astype(jnp.int16)
        for sz in (512, 256, 128, 64, 32, 16):
            m = m[:sz] + m[sz:]
        cnt = jnp.sum(m.astype(jnp.int32), axis=0, keepdims=True)
        r = jnp.where(cnt >= KK, cand, r)
    key_prefix = lax.shift_left(r - 32768, 16)
    thr = _key_to_float(key_prefix)

    p = jnp.where(at >= thr, jnp.exp(at), 0.0)
    vo = jnp.concatenate([v, jnp.ones((1, N), jnp.float32)], axis=0)
    for ti in range(GROUP):
        psl = p[:, ti * HW:(ti + 1) * HW]
        osl = lax.dot_general(vo, psl, (((1,), (0,)), ((), ())),
                              preferred_element_type=jnp.float32)
        o_acc[ti, h] = osl[:CP, :] / osl[CP:CP + 1, :]

    @pl.when(h == HEADS - 1)
    def _project():
        o2 = o_acc[...].reshape(C, HW)
        out_ref[0] = x2_ref[0] + lax.dot_general(
            wp_ref[...], o2, (((1,), (0,)), ((), ())),
            preferred_element_type=jnp.float32)


def kernel(x, W_qkv, W_dw, temperature, W_proj):
    x2 = x.reshape(B, C, HW)
    xh4 = x.reshape(B, GROUP, HEADS, CP, HW)

    y = pl.pallas_call(
        _fused_kernel,
        grid=(B, HEADS),
        in_specs=[
            pl.BlockSpec((1, GROUP, 1, CP, HW), lambda b, h: (b, 0, h, 0, 0)),
            pl.BlockSpec((1, C, HW), lambda b, h: (b, 0, 0)),
            pl.BlockSpec(memory_space=pltpu.SMEM),
            pl.BlockSpec(memory_space=pltpu.SMEM),
            pl.BlockSpec(memory_space=pltpu.SMEM),
            pl.BlockSpec((C, C), lambda b, h: (0, 0)),
        ],
        out_specs=pl.BlockSpec((1, C, HW), lambda b, h: (b, 0, 0)),
        out_shape=jax.ShapeDtypeStruct((B, C, HW), jnp.float32),
        scratch_shapes=[pltpu.VMEM((GROUP, HEADS, CP, HW), jnp.float32)],
    )(xh4, x2, W_qkv, W_dw, temperature, W_proj)
    return y.reshape(B, C, HH, WW)
